# baseline (speedup 1.0000x reference)
import functools

import jax
import jax.numpy as jnp
from jax import lax
from jax.experimental import pallas as pl
from jax.experimental.pallas import tpu as pltpu
from jax.experimental.pallas import tpu_sc as plsc

DIM = 128
NIDX = 1024 * 50
NWORKERS = 32
PER_W = NIDX // NWORKERS
CHUNK = 64
NCHUNKS = PER_W // CHUNK
OUT_ROW = DIM * 4

_RSQRT_MAGIC = jnp.int32(0x5F3759DF)


def _body(x_hbm, scale_hbm, r_hbm, i_hbm, j_hbm, k_hbm, out_hbm,
          idx_v, scale_v, rv, iv, jv, kv, out_v, gsem):
    nc = 2
    wid = lax.axis_index("s") * nc + lax.axis_index("c")
    base = wid * PER_W

    pltpu.sync_copy(x_hbm.at[pl.ds(base, PER_W)], idx_v)
    pltpu.sync_copy(scale_hbm, scale_v)

    lane4 = lax.iota(jnp.int32, 16) * 4
    scale_regs = [scale_v[pl.ds(16 * g, 16)] for g in range(8)]

    def chunk_body(c, carry):
        idx_ref = idx_v.at[pl.ds(c * CHUNK, CHUNK)]
        d1 = pltpu.async_copy(r_hbm.at[idx_ref], rv, gsem)
        d2 = pltpu.async_copy(i_hbm.at[idx_ref], iv, gsem)
        d3 = pltpu.async_copy(j_hbm.at[idx_ref], jv, gsem)
        d4 = pltpu.async_copy(k_hbm.at[idx_ref], kv, gsem)
        d1.wait()
        d2.wait()
        d3.wait()
        d4.wait()

        def row_body(b, rcarry):
            out_base = b * OUT_ROW
            for g in range(8):
                sl = pl.ds(g * 16, 16)
                rr = rv[b, sl]
                ii = iv[b, sl] * scale_regs[g]
                jj = jv[b, sl] * scale_regs[g]
                kk = kv[b, sl] * scale_regs[g]
                s = rr * rr + ii * ii + jj * jj + kk * kk + 1e-6
                y = plsc.bitcast(
                    _RSQRT_MAGIC - lax.shift_right_logical(
                        plsc.bitcast(s, jnp.int32), 1),
                    jnp.float32)
                xh = s * 0.5
                y = y * (1.5 - xh * y * y)
                y = y * (1.5 - xh * y * y)
                y = y * (1.5 - xh * y * y)
                col = lane4 + (out_base + g * 64)
                plsc.store_scatter(out_v, [col], rr * y)
                plsc.store_scatter(out_v, [col + 1], ii * y)
                plsc.store_scatter(out_v, [col + 2], jj * y)
                plsc.store_scatter(out_v, [col + 3], kk * y)
            return rcarry

        lax.fori_loop(0, CHUNK, row_body, 0)
        pltpu.sync_copy(
            out_v, out_hbm.at[pl.ds((base + c * CHUNK) * OUT_ROW,
                                    CHUNK * OUT_ROW)])
        return carry

    lax.fori_loop(0, NCHUNKS, chunk_body, 0)


_qembed = functools.partial(
    pl.kernel,
    out_type=jax.ShapeDtypeStruct((NIDX * OUT_ROW,), jnp.float32),
    mesh=plsc.VectorSubcoreMesh(core_axis_name="c", subcore_axis_name="s"),
    compiler_params=pltpu.CompilerParams(needs_layout_passes=False),
    scratch_types=[
        pltpu.VMEM((PER_W,), jnp.int32),
        pltpu.VMEM((DIM,), jnp.float32),
        pltpu.VMEM((CHUNK, DIM), jnp.float32),
        pltpu.VMEM((CHUNK, DIM), jnp.float32),
        pltpu.VMEM((CHUNK, DIM), jnp.float32),
        pltpu.VMEM((CHUNK, DIM), jnp.float32),
        pltpu.VMEM((CHUNK * OUT_ROW,), jnp.float32),
        pltpu.SemaphoreType.DMA,
    ],
)(_body)


def kernel(x, scalar, vector_i, vector_j, vector_k):
    dim = scalar.shape[1]
    scale = 1.0 / (10000.0 ** (jnp.arange(dim, dtype=jnp.float32) / dim))
    xf = x.reshape(-1).astype(jnp.int32)
    out = _qembed(xf, scale.astype(jnp.float32), scalar,
                  vector_i, vector_j, vector_k)
    return out.reshape(x.shape[0], x.shape[1], dim, 4)


# speedup vs baseline: 1.0505x; 1.0505x over previous
import functools

import jax
import jax.numpy as jnp
from jax import lax
from jax.experimental import pallas as pl
from jax.experimental.pallas import tpu as pltpu
from jax.experimental.pallas import tpu_sc as plsc

DIM = 128
NIDX = 1024 * 50
NWORKERS = 32
PER_W = NIDX // NWORKERS
CHUNK = 64
NCHUNKS = PER_W // CHUNK
OUT_ROW = DIM * 4

_RSQRT_MAGIC = jnp.int32(0x5F3759DF)


def _body(x_hbm, scale_hbm, r_hbm, i_hbm, j_hbm, k_hbm, out_hbm,
          idx_v, scale_v, rv, iv, jv, kv, out_v, gsem):
    nc = 2
    wid = lax.axis_index("s") * nc + lax.axis_index("c")
    base = wid * PER_W

    pltpu.sync_copy(x_hbm.at[pl.ds(base, PER_W)], idx_v)
    pltpu.sync_copy(scale_hbm, scale_v)

    lane4 = lax.iota(jnp.int32, 16) * 4
    scale_regs = [scale_v[pl.ds(16 * g, 16)] for g in range(8)]

    def chunk_body(c, carry):
        idx_ref = idx_v.at[pl.ds(c * CHUNK, CHUNK)]
        d1 = pltpu.async_copy(r_hbm.at[idx_ref], rv, gsem)
        d2 = pltpu.async_copy(i_hbm.at[idx_ref], iv, gsem)
        d3 = pltpu.async_copy(j_hbm.at[idx_ref], jv, gsem)
        d4 = pltpu.async_copy(k_hbm.at[idx_ref], kv, gsem)
        d1.wait()
        d2.wait()
        d3.wait()
        d4.wait()

        @plsc.parallel_loop(0, CHUNK, unroll=4)
        def row_body(b):
            out_base = b * OUT_ROW
            for g in range(8):
                sl = pl.ds(g * 16, 16)
                rr = rv[b, sl]
                ii = iv[b, sl] * scale_regs[g]
                jj = jv[b, sl] * scale_regs[g]
                kk = kv[b, sl] * scale_regs[g]
                s = rr * rr + ii * ii + jj * jj + kk * kk + 1e-6
                y = plsc.bitcast(
                    _RSQRT_MAGIC - lax.shift_right_logical(
                        plsc.bitcast(s, jnp.int32), 1),
                    jnp.float32)
                xh = s * 0.5
                y = y * (1.5 - xh * y * y)
                y = y * (1.5 - xh * y * y)
                y = y * (1.5 - xh * y * y)
                col = lane4 + (out_base + g * 64)
                plsc.store_scatter(out_v, [col], rr * y)
                plsc.store_scatter(out_v, [col + 1], ii * y)
                plsc.store_scatter(out_v, [col + 2], jj * y)
                plsc.store_scatter(out_v, [col + 3], kk * y)

        pltpu.sync_copy(
            out_v, out_hbm.at[pl.ds((base + c * CHUNK) * OUT_ROW,
                                    CHUNK * OUT_ROW)])
        return carry

    lax.fori_loop(0, NCHUNKS, chunk_body, 0)


_qembed = functools.partial(
    pl.kernel,
    out_type=jax.ShapeDtypeStruct((NIDX * OUT_ROW,), jnp.float32),
    mesh=plsc.VectorSubcoreMesh(core_axis_name="c", subcore_axis_name="s"),
    compiler_params=pltpu.CompilerParams(needs_layout_passes=False),
    scratch_types=[
        pltpu.VMEM((PER_W,), jnp.int32),
        pltpu.VMEM((DIM,), jnp.float32),
        pltpu.VMEM((CHUNK, DIM), jnp.float32),
        pltpu.VMEM((CHUNK, DIM), jnp.float32),
        pltpu.VMEM((CHUNK, DIM), jnp.float32),
        pltpu.VMEM((CHUNK, DIM), jnp.float32),
        pltpu.VMEM((CHUNK * OUT_ROW,), jnp.float32),
        pltpu.SemaphoreType.DMA,
    ],
)(_body)


def kernel(x, scalar, vector_i, vector_j, vector_k):
    dim = scalar.shape[1]
    scale = 1.0 / (10000.0 ** (jnp.arange(dim, dtype=jnp.float32) / dim))
    xf = x.reshape(-1).astype(jnp.int32)
    out = _qembed(xf, scale.astype(jnp.float32), scalar,
                  vector_i, vector_j, vector_k)
    return out.reshape(x.shape[0], x.shape[1], dim, 4)


# speedup vs baseline: 22.0424x; 20.9825x over previous
import functools

import jax
import jax.numpy as jnp
from jax import lax
from jax.experimental import pallas as pl
from jax.experimental.pallas import tpu as pltpu
from jax.experimental.pallas import tpu_sc as plsc

DIM = 128
NIDX = 1024 * 50
NWORKERS = 32
PER_W = NIDX // NWORKERS
CHUNK = 64
NCHUNKS = PER_W // CHUNK
OUT_ROW = DIM * 4

_RSQRT_MAGIC = 0x5F3759DF


def _body(x_hbm, scale_hbm, r_hbm, i_hbm, j_hbm, k_hbm, out_hbm,
          idx_v, scale_v, rv, iv, jv, kv, out_v, gsem):
    nc = 2
    wid = lax.axis_index("s") * nc + lax.axis_index("c")
    base = wid * PER_W

    pltpu.sync_copy(x_hbm.at[pl.ds(base, PER_W)], idx_v)
    pltpu.sync_copy(scale_hbm, scale_v)

    scale_regs = [scale_v[pl.ds(16 * g, 16)] for g in range(8)]

    def chunk_body(c, carry):
        idx_ref = idx_v.at[pl.ds(c * CHUNK, CHUNK)]
        d1 = pltpu.async_copy(r_hbm.at[idx_ref], rv, gsem)
        d2 = pltpu.async_copy(i_hbm.at[idx_ref], iv, gsem)
        d3 = pltpu.async_copy(j_hbm.at[idx_ref], jv, gsem)
        d4 = pltpu.async_copy(k_hbm.at[idx_ref], kv, gsem)
        d1.wait()
        d2.wait()
        d3.wait()
        d4.wait()

        @plsc.parallel_loop(0, CHUNK, unroll=4)
        def row_body(b):
            out_base = b * OUT_ROW
            for g in range(8):
                sl = pl.ds(g * 16, 16)
                rr = rv[b, sl]
                ii = iv[b, sl] * scale_regs[g]
                jj = jv[b, sl] * scale_regs[g]
                kk = kv[b, sl] * scale_regs[g]
                s = rr * rr + ii * ii + jj * jj + kk * kk + 1e-6
                y = plsc.bitcast(
                    _RSQRT_MAGIC - lax.shift_right_logical(
                        plsc.bitcast(s, jnp.int32), 1),
                    jnp.float32)
                xh = s * 0.5
                y = y * (1.5 - xh * y * y)
                y = y * (1.5 - xh * y * y)
                y = y * (1.5 - xh * y * y)
                out_v[pl.ds(out_base + g * 16, 16)] = rr * y
                out_v[pl.ds(out_base + DIM + g * 16, 16)] = ii * y
                out_v[pl.ds(out_base + 2 * DIM + g * 16, 16)] = jj * y
                out_v[pl.ds(out_base + 3 * DIM + g * 16, 16)] = kk * y

        pltpu.sync_copy(
            out_v, out_hbm.at[pl.ds((base + c * CHUNK) * OUT_ROW,
                                    CHUNK * OUT_ROW)])
        return carry

    lax.fori_loop(0, NCHUNKS, chunk_body, 0)


_qembed = functools.partial(
    pl.kernel,
    out_type=jax.ShapeDtypeStruct((NIDX * OUT_ROW,), jnp.float32),
    mesh=plsc.VectorSubcoreMesh(core_axis_name="c", subcore_axis_name="s"),
    compiler_params=pltpu.CompilerParams(needs_layout_passes=False),
    scratch_types=[
        pltpu.VMEM((PER_W,), jnp.int32),
        pltpu.VMEM((DIM,), jnp.float32),
        pltpu.VMEM((CHUNK, DIM), jnp.float32),
        pltpu.VMEM((CHUNK, DIM), jnp.float32),
        pltpu.VMEM((CHUNK, DIM), jnp.float32),
        pltpu.VMEM((CHUNK, DIM), jnp.float32),
        pltpu.VMEM((CHUNK * OUT_ROW,), jnp.float32),
        pltpu.SemaphoreType.DMA,
    ],
)(_body)


def kernel(x, scalar, vector_i, vector_j, vector_k):
    dim = scalar.shape[1]
    scale = 1.0 / (10000.0 ** (jnp.arange(dim, dtype=jnp.float32) / dim))
    xf = x.reshape(-1).astype(jnp.int32)
    out = _qembed(xf, scale.astype(jnp.float32), scalar,
                  vector_i, vector_j, vector_k)
    out = out.reshape(x.shape[0], x.shape[1], 4, dim)
    return jnp.swapaxes(out, -1, -2)


# speedup vs baseline: 35.2717x; 1.6002x over previous
import functools

import jax
import jax.numpy as jnp
from jax import lax
from jax.experimental import pallas as pl
from jax.experimental.pallas import tpu as pltpu
from jax.experimental.pallas import tpu_sc as plsc

DIM = 128
NIDX = 1024 * 50
NWORKERS = 32
PER_W = NIDX // NWORKERS
CHUNK = 32
NCHUNKS = PER_W // CHUNK
OUT_ROW = DIM * 4

_RSQRT_MAGIC = 0x5F3759DF


def _body(x_hbm, scale_hbm, r_hbm, i_hbm, j_hbm, k_hbm, out_hbm,
          idx_v, scale_v,
          rv0, iv0, jv0, kv0, ov0,
          rv1, iv1, jv1, kv1, ov1,
          gsem, osem):
    nc = 2
    wid = lax.axis_index("s") * nc + lax.axis_index("c")
    base = wid * PER_W

    pltpu.sync_copy(x_hbm.at[pl.ds(base, PER_W)], idx_v)
    pltpu.sync_copy(scale_hbm, scale_v)

    scale_regs = [scale_v[pl.ds(16 * g, 16)] for g in range(8)]
    tabs = (r_hbm, i_hbm, j_hbm, k_hbm)
    bufs = ((rv0, iv0, jv0, kv0), (rv1, iv1, jv1, kv1))
    outs = (ov0, ov1)

    def fire_gathers(c, s):
        idx_ref = idx_v.at[pl.ds(c * CHUNK, CHUNK)]
        for t, b in zip(tabs, bufs[s]):
            pltpu.async_copy(t.at[idx_ref], b, gsem)

    def drain_gathers(s):
        idx_ref = idx_v.at[pl.ds(0, CHUNK)]
        for t, b in zip(tabs, bufs[s]):
            pltpu.make_async_copy(t.at[idx_ref], b, gsem).wait()

    def drain_out(s):
        pltpu.make_async_copy(
            outs[s], out_hbm.at[pl.ds(0, CHUNK * OUT_ROW)], osem).wait()

    fire_gathers(0, 0)

    def super_body(c2, carry):
        for s in range(2):
            c = c2 * 2 + s
            rv, iv, jv, kv = bufs[s]
            ov = outs[s]
            drain_gathers(s)

            @pl.when(c + 1 < NCHUNKS)
            def _():
                fire_gathers(c + 1, 1 - s)

            @pl.when(c >= 2)
            def _():
                drain_out(s)

            @plsc.parallel_loop(0, CHUNK, unroll=4)
            def row_body(b):
                out_base = b * OUT_ROW
                for g in range(8):
                    sl = pl.ds(g * 16, 16)
                    rr = rv[b, sl]
                    ii = iv[b, sl] * scale_regs[g]
                    jj = jv[b, sl] * scale_regs[g]
                    kk = kv[b, sl] * scale_regs[g]
                    sq = rr * rr + ii * ii + jj * jj + kk * kk + 1e-6
                    y = plsc.bitcast(
                        _RSQRT_MAGIC - lax.shift_right_logical(
                            plsc.bitcast(sq, jnp.int32), 1),
                        jnp.float32)
                    xh = sq * 0.5
                    y = y * (1.5 - xh * y * y)
                    y = y * (1.5 - xh * y * y)
                    y = y * (1.5 - xh * y * y)
                    ov[pl.ds(out_base + g * 16, 16)] = rr * y
                    ov[pl.ds(out_base + DIM + g * 16, 16)] = ii * y
                    ov[pl.ds(out_base + 2 * DIM + g * 16, 16)] = jj * y
                    ov[pl.ds(out_base + 3 * DIM + g * 16, 16)] = kk * y

            pltpu.async_copy(
                ov, out_hbm.at[pl.ds((base + c * CHUNK) * OUT_ROW,
                                     CHUNK * OUT_ROW)], osem)
        return carry

    lax.fori_loop(0, NCHUNKS // 2, super_body, 0)
    drain_out(0)
    drain_out(1)


_qembed = functools.partial(
    pl.kernel,
    out_type=jax.ShapeDtypeStruct((NIDX * OUT_ROW,), jnp.float32),
    mesh=plsc.VectorSubcoreMesh(core_axis_name="c", subcore_axis_name="s"),
    compiler_params=pltpu.CompilerParams(needs_layout_passes=False),
    scratch_types=(
        [pltpu.VMEM((PER_W,), jnp.int32), pltpu.VMEM((DIM,), jnp.float32)]
        + [pltpu.VMEM((CHUNK, DIM), jnp.float32)] * 4
        + [pltpu.VMEM((CHUNK * OUT_ROW,), jnp.float32)]
        + [pltpu.VMEM((CHUNK, DIM), jnp.float32)] * 4
        + [pltpu.VMEM((CHUNK * OUT_ROW,), jnp.float32)]
        + [pltpu.SemaphoreType.DMA, pltpu.SemaphoreType.DMA]
    ),
)(_body)


def kernel(x, scalar, vector_i, vector_j, vector_k):
    dim = scalar.shape[1]
    scale = 1.0 / (10000.0 ** (jnp.arange(dim, dtype=jnp.float32) / dim))
    xf = x.reshape(-1).astype(jnp.int32)
    out = _qembed(xf, scale.astype(jnp.float32), scalar,
                  vector_i, vector_j, vector_k)
    out = out.reshape(x.shape[0], x.shape[1], 4, dim)
    return jnp.swapaxes(out, -1, -2)


# speedup vs baseline: 35.3198x; 1.0014x over previous
import functools

import jax
import jax.numpy as jnp
from jax import lax
from jax.experimental import pallas as pl
from jax.experimental.pallas import tpu as pltpu
from jax.experimental.pallas import tpu_sc as plsc

DIM = 128
NIDX = 1024 * 50
NWORKERS = 32
PER_W = NIDX // NWORKERS
CHUNK = 32
NCHUNKS = PER_W // CHUNK
OUT_ROW = DIM * 4

_RSQRT_MAGIC = 0x5F3759DF


def _body(x_hbm, scale_hbm, r_hbm, i_hbm, j_hbm, k_hbm, out_hbm,
          idx_v, scale_v,
          rv0, iv0, jv0, kv0, ov0,
          rv1, iv1, jv1, kv1, ov1,
          gsem, osem):
    nc = 2
    wid = lax.axis_index("s") * nc + lax.axis_index("c")
    base = wid * PER_W

    pltpu.sync_copy(x_hbm.at[pl.ds(base, PER_W)], idx_v)
    pltpu.sync_copy(scale_hbm, scale_v)

    scale_regs = [scale_v[pl.ds(16 * g, 16)] for g in range(8)]
    tabs = (r_hbm, i_hbm, j_hbm, k_hbm)
    bufs = ((rv0, iv0, jv0, kv0), (rv1, iv1, jv1, kv1))
    outs = (ov0, ov1)

    def fire_gathers(c, s):
        idx_ref = idx_v.at[pl.ds(c * CHUNK, CHUNK)]
        for t, b in zip(tabs, bufs[s]):
            pltpu.async_copy(t.at[idx_ref], b, gsem)

    def drain_gathers(s):
        idx_ref = idx_v.at[pl.ds(0, CHUNK)]
        for t, b in zip(tabs, bufs[s]):
            pltpu.make_async_copy(t.at[idx_ref], b, gsem).wait()

    def drain_out(s):
        pltpu.make_async_copy(
            outs[s], out_hbm.at[pl.ds(0, CHUNK * OUT_ROW)], osem).wait()

    fire_gathers(0, 0)

    def super_body(c2, carry):
        for s in range(2):
            c = c2 * 2 + s
            rv, iv, jv, kv = bufs[s]
            ov = outs[s]
            drain_gathers(s)

            @pl.when(c + 1 < NCHUNKS)
            def _():
                fire_gathers(c + 1, 1 - s)

            @pl.when(c >= 2)
            def _():
                drain_out(s)

            @plsc.parallel_loop(0, CHUNK, unroll=8)
            def row_body(b):
                out_base = b * OUT_ROW
                for g in range(8):
                    sl = pl.ds(g * 16, 16)
                    rr = rv[b, sl]
                    ii = iv[b, sl] * scale_regs[g]
                    jj = jv[b, sl] * scale_regs[g]
                    kk = kv[b, sl] * scale_regs[g]
                    sq = rr * rr + ii * ii + jj * jj + kk * kk + 1e-6
                    y = plsc.bitcast(
                        _RSQRT_MAGIC - lax.shift_right_logical(
                            plsc.bitcast(sq, jnp.int32), 1),
                        jnp.float32)
                    xh = sq * 0.5
                    y = y * (1.5 - xh * y * y)
                    y = y * (1.5 - xh * y * y)
                    ov[pl.ds(out_base + g * 16, 16)] = rr * y
                    ov[pl.ds(out_base + DIM + g * 16, 16)] = ii * y
                    ov[pl.ds(out_base + 2 * DIM + g * 16, 16)] = jj * y
                    ov[pl.ds(out_base + 3 * DIM + g * 16, 16)] = kk * y

            pltpu.async_copy(
                ov, out_hbm.at[pl.ds((base + c * CHUNK) * OUT_ROW,
                                     CHUNK * OUT_ROW)], osem)
        return carry

    lax.fori_loop(0, NCHUNKS // 2, super_body, 0)
    drain_out(0)
    drain_out(1)


_qembed = functools.partial(
    pl.kernel,
    out_type=jax.ShapeDtypeStruct((NIDX * OUT_ROW,), jnp.float32),
    mesh=plsc.VectorSubcoreMesh(core_axis_name="c", subcore_axis_name="s"),
    compiler_params=pltpu.CompilerParams(needs_layout_passes=False),
    scratch_types=(
        [pltpu.VMEM((PER_W,), jnp.int32), pltpu.VMEM((DIM,), jnp.float32)]
        + [pltpu.VMEM((CHUNK, DIM), jnp.float32)] * 4
        + [pltpu.VMEM((CHUNK * OUT_ROW,), jnp.float32)]
        + [pltpu.VMEM((CHUNK, DIM), jnp.float32)] * 4
        + [pltpu.VMEM((CHUNK * OUT_ROW,), jnp.float32)]
        + [pltpu.SemaphoreType.DMA, pltpu.SemaphoreType.DMA]
    ),
)(_body)


def kernel(x, scalar, vector_i, vector_j, vector_k):
    dim = scalar.shape[1]
    scale = 1.0 / (10000.0 ** (jnp.arange(dim, dtype=jnp.float32) / dim))
    xf = x.reshape(-1).astype(jnp.int32)
    out = _qembed(xf, scale.astype(jnp.float32), scalar,
                  vector_i, vector_j, vector_k)
    out = out.reshape(x.shape[0], x.shape[1], 4, dim)
    return jnp.swapaxes(out, -1, -2)


# speedup vs baseline: 38.2882x; 1.0840x over previous
import functools

import jax
import jax.numpy as jnp
from jax import lax
from jax.experimental import pallas as pl
from jax.experimental.pallas import tpu as pltpu
from jax.experimental.pallas import tpu_sc as plsc

DIM = 128
NIDX = 1024 * 50
NWORKERS = 32
PER_W = NIDX // NWORKERS
CHUNK = 40
NCHUNKS = PER_W // CHUNK
OUT_ROW = DIM * 4

_RSQRT_MAGIC = 0x5F3759DF


def _body(x_hbm, scale_hbm, r_hbm, i_hbm, j_hbm, k_hbm, out_hbm,
          idx_v, scale_v, qv0, ov0, qv1, ov1, gsem, osem):
    nc = 2
    wid = lax.axis_index("s") * nc + lax.axis_index("c")
    base = wid * PER_W

    pltpu.sync_copy(x_hbm.at[pl.ds(base, PER_W)], idx_v)
    pltpu.sync_copy(scale_hbm, scale_v)

    scale_regs = [scale_v[pl.ds(16 * g, 16)] for g in range(8)]
    tabs = (r_hbm, i_hbm, j_hbm, k_hbm)
    qvs = (qv0, qv1)
    outs = (ov0, ov1)

    def fire_gathers(c, s):
        idx_ref = idx_v.at[pl.ds(c * CHUNK, CHUNK)]
        for t, tab in enumerate(tabs):
            pltpu.async_copy(
                tab.at[idx_ref], qvs[s].at[pl.ds(t * CHUNK, CHUNK)], gsem)

    def drain_gathers(s):
        pltpu.make_async_copy(
            r_hbm.at[pl.ds(0, 4 * CHUNK)], qvs[s], gsem).wait()

    def drain_out(s):
        pltpu.make_async_copy(
            outs[s], out_hbm.at[pl.ds(0, CHUNK * OUT_ROW)], osem).wait()

    fire_gathers(0, 0)

    def super_body(c2, carry):
        for s in range(2):
            c = c2 * 2 + s
            qv = qvs[s]
            ov = outs[s]
            drain_gathers(s)

            @pl.when(c + 1 < NCHUNKS)
            def _():
                fire_gathers(c + 1, 1 - s)

            @pl.when(c >= 2)
            def _():
                drain_out(s)

            @plsc.parallel_loop(0, CHUNK, unroll=8)
            def row_body(b):
                out_base = b * OUT_ROW
                for g in range(8):
                    sl = pl.ds(g * 16, 16)
                    rr = qv[b, sl]
                    ii = qv[CHUNK + b, sl] * scale_regs[g]
                    jj = qv[2 * CHUNK + b, sl] * scale_regs[g]
                    kk = qv[3 * CHUNK + b, sl] * scale_regs[g]
                    sq = rr * rr + ii * ii + jj * jj + kk * kk + 1e-6
                    y = plsc.bitcast(
                        _RSQRT_MAGIC - lax.shift_right_logical(
                            plsc.bitcast(sq, jnp.int32), 1),
                        jnp.float32)
                    xh = sq * 0.5
                    y = y * (1.5 - xh * y * y)
                    y = y * (1.5 - xh * y * y)
                    ov[pl.ds(out_base + g * 16, 16)] = rr * y
                    ov[pl.ds(out_base + DIM + g * 16, 16)] = ii * y
                    ov[pl.ds(out_base + 2 * DIM + g * 16, 16)] = jj * y
                    ov[pl.ds(out_base + 3 * DIM + g * 16, 16)] = kk * y

            pltpu.async_copy(
                ov, out_hbm.at[pl.ds((base + c * CHUNK) * OUT_ROW,
                                     CHUNK * OUT_ROW)], osem)
        return carry

    lax.fori_loop(0, NCHUNKS // 2, super_body, 0)
    drain_out(0)
    drain_out(1)


_qembed = functools.partial(
    pl.kernel,
    out_type=jax.ShapeDtypeStruct((NIDX * OUT_ROW,), jnp.float32),
    mesh=plsc.VectorSubcoreMesh(core_axis_name="c", subcore_axis_name="s"),
    compiler_params=pltpu.CompilerParams(needs_layout_passes=False),
    scratch_types=(
        [pltpu.VMEM((PER_W,), jnp.int32), pltpu.VMEM((DIM,), jnp.float32)]
        + [pltpu.VMEM((4 * CHUNK, DIM), jnp.float32),
           pltpu.VMEM((CHUNK * OUT_ROW,), jnp.float32)] * 2
        + [pltpu.SemaphoreType.DMA, pltpu.SemaphoreType.DMA]
    ),
)(_body)


def kernel(x, scalar, vector_i, vector_j, vector_k):
    dim = scalar.shape[1]
    scale = 1.0 / (10000.0 ** (jnp.arange(dim, dtype=jnp.float32) / dim))
    xf = x.reshape(-1).astype(jnp.int32)
    out = _qembed(xf, scale.astype(jnp.float32), scalar,
                  vector_i, vector_j, vector_k)
    out = out.reshape(x.shape[0], x.shape[1], 4, dim)
    return jnp.swapaxes(out, -1, -2)
